# baseline (speedup 1.0000x reference)
import jax
import jax.numpy as jnp
from jax import lax
from jax.experimental import pallas as pl
from jax.experimental.pallas import tpu as pltpu
from jax.experimental.pallas import tpu_sc as plsc

N = 10000
E = 320000
D = 128
C = 40
B = 2048

NC = 2
NS = 16
L = 16

HALF = N // NC
EPT = E // NS
EPC = 2048
PASS_SIZES = [EPC] * (EPT // EPC) + [EPT - (EPT // EPC) * EPC]
KC = 64
TROWS = 64
SLOTS = 2176
DUMMY = B
DW = 128
DROWS = 32
BT = B // NS
XPT = 320
MPT = 624


def _sc_kernel(x_hbm, ei_hbm, node_hbm,
               acc_out, deg_out,
               edb_s0, edb_s1, edb_d0, edb_d1, node_f, smap_v, deg_l,
               csrc2, cslot2, rows_a, rows_b,
               iota_v, slotidx_v, degout_v,
               x_sh, acc_sh, deg_sh, smap_sh,
               gsem_a, gsem_b, esem, xsem):
  cid = lax.axis_index("c")
  sid = lax.axis_index("s")
  ebase = sid * EPT
  xbase = cid * HALF

  sc_setup = jax.named_scope("sc_setup")
  sc_setup.__enter__()
  edbs = (edb_s0, edb_s1)
  edbd = (edb_d0, edb_d1)
  pltpu.async_copy(ei_hbm.at[pl.ds(ebase, EPC)], edb_s0, esem)
  pltpu.async_copy(ei_hbm.at[pl.ds(E + ebase, EPC)], edb_d0, esem)
  @pl.when(sid < NS - 1)
  def _xstage():
    pltpu.async_copy(x_hbm.at[pl.ds(xbase + sid * XPT, XPT)],
                     x_sh.at[pl.ds(sid * XPT, XPT)], xsem)
  @pl.when(sid == NS - 1)
  def _xstage_last():
    pltpu.async_copy(
        x_hbm.at[pl.ds(xbase + (NS - 1) * XPT, HALF - (NS - 1) * XPT)],
        x_sh.at[pl.ds((NS - 1) * XPT, HALF - (NS - 1) * XPT)], xsem)

  zeros_f = jnp.zeros((L,), jnp.float32)
  ones_f = jnp.ones((L,), jnp.float32)
  iota = lax.iota(jnp.int32, L)

  def zrows(i, _):
    for j in range(D // L):
      rows_a[i, pl.ds(j * L, L)] = zeros_f
    return 0
  lax.fori_loop(0, KC, zrows, 0)

  def zdl(i, _):
    for j in range(DW // L):
      deg_l[i, pl.ds(j * L, L)] = zeros_f
    return 0
  lax.fori_loop(0, DROWS, zdl, 0)

  for g in range(DROWS // L):
    iota_v[pl.ds(g * L, L)] = g * L + iota

  zbase = sid * (SLOTS // NS)
  for q in range(SLOTS // NS // KC):
    pltpu.sync_copy(rows_a, acc_sh.at[pl.ds(zbase + q * KC, KC)])
  _zr = SLOTS // NS - (SLOTS // NS // KC) * KC
  pltpu.sync_copy(rows_a.at[pl.ds(0, _zr)],
                  acc_sh.at[pl.ds(zbase + SLOTS // NS - _zr, _zr)])

  @pl.when(sid == 0)
  def _zdeg():
    pltpu.sync_copy(rows_a.at[pl.ds(0, DROWS)], deg_sh)
  pltpu.sync_copy(node_hbm, node_f)
  mlo = sid * MPT
  msz = jnp.where(sid == NS - 1, N - (NS - 1) * MPT, MPT)
  neg = jnp.full((L,), -1, jnp.int32)
  def zmap(i, _):
    smap_v[pl.ds(mlo + i * L, L)] = neg
    return 0
  lax.fori_loop(0, msz // L, zmap, 0)
  def setmap(i, _):
    idx = node_f[pl.ds(i * L, L)]
    inrange = (idx >= mlo) & (idx < mlo + msz)
    plsc.store_scatter(smap_v, [idx], i * L + iota, mask=inrange)
    return 0
  lax.fori_loop(0, B // L, setmap, 0)
  pltpu.sync_copy(smap_v.at[pl.ds(mlo, 624)], smap_sh.at[pl.ds(mlo, 624)])
  @pl.when(sid == NS - 1)
  def _pub_tail():
    pltpu.sync_copy(smap_v.at[pl.ds(N - L, L)], smap_sh.at[pl.ds(N - L, L)])

  @pl.when(sid < NS - 1)
  def _xwait():
    pltpu.make_async_copy(x_hbm.at[pl.ds(xbase + sid * XPT, XPT)],
                          x_sh.at[pl.ds(sid * XPT, XPT)], xsem).wait()
  @pl.when(sid == NS - 1)
  def _xwait_last():
    pltpu.make_async_copy(
        x_hbm.at[pl.ds(xbase + (NS - 1) * XPT, HALF - (NS - 1) * XPT)],
        x_sh.at[pl.ds((NS - 1) * XPT, HALF - (NS - 1) * XPT)], xsem).wait()
  sc_setup.__exit__(None, None, None)
  sc_bar1 = jax.named_scope("sc_bar1")
  sc_bar1.__enter__()
  plsc.subcore_barrier()
  pltpu.sync_copy(smap_sh, smap_v)
  sc_bar1.__exit__(None, None, None)

  def flush(f, mch):
    @pl.when(mch > f)
    def _prime():
      pltpu.async_copy(x_sh.at[csrc2.at[f % TROWS]], rows_a, gsem_a)
    def rowpair(i, _):
      ch0 = f + 2 * i
      ch1 = ch0 + 1
      r0 = ch0 % TROWS
      r1 = ch1 % TROWS
      pltpu.make_async_copy(x_sh.at[csrc2.at[r0]], rows_a, gsem_a).wait()
      @pl.when(ch1 < mch)
      def _g1():
        pltpu.async_copy(x_sh.at[csrc2.at[r1]], rows_b, gsem_b)
      pltpu.sync_copy(rows_a, acc_sh.at[cslot2.at[r0]], add=True)
      @pl.when(ch1 < mch)
      def _s1():
        pltpu.make_async_copy(x_sh.at[csrc2.at[r1]], rows_b, gsem_b).wait()
        @pl.when(ch1 + 1 < mch)
        def _g2():
          pltpu.async_copy(x_sh.at[csrc2.at[(ch1 + 1) % TROWS]],
                           rows_a, gsem_a)
        pltpu.sync_copy(rows_b, acc_sh.at[cslot2.at[r1]], add=True)
      return 0
    lax.fori_loop(0, (mch - f + 1) // 2, rowpair, 0)

  sc_work = jax.named_scope("sc_work")
  sc_work.__enter__()
  m = jnp.int32(0)
  f = jnp.int32(0)
  for p, psz in enumerate(PASS_SIZES):
    par = p % 2
    eoff = ebase + p * EPC
    pltpu.make_async_copy(ei_hbm.at[pl.ds(eoff, psz)],
                          edbs[par].at[pl.ds(0, psz)], esem).wait()
    pltpu.make_async_copy(ei_hbm.at[pl.ds(E + eoff, psz)],
                          edbd[par].at[pl.ds(0, psz)], esem).wait()
    if p + 1 < len(PASS_SIZES):
      nsz = PASS_SIZES[p + 1]
      pltpu.async_copy(ei_hbm.at[pl.ds(eoff + EPC, nsz)],
                       edbs[1 - par].at[pl.ds(0, nsz)], esem)
      pltpu.async_copy(ei_hbm.at[pl.ds(E + eoff + EPC, nsz)],
                       edbd[1 - par].at[pl.ds(0, nsz)], esem)

    def half(g, m, par):
      d = edbd[par][pl.ds(g * L, L)]
      s = edbs[par][pl.ds(g * L, L)]
      slot = plsc.load_gather(smap_v, [d])
      sl = s - xbase
      msk = (slot >= 0) & (sl >= 0) & (sl < HALF)
      mv = jnp.where(msk, 1, 0).astype(jnp.int32)
      inc = plsc.cumsum(mv)
      return d, s, slot, sl, msk, inc

    def emit(m, slot, sl, msk, inc):
      pos = m + inc - 1
      row = (pos // KC) % TROWS
      col = pos % KC
      plsc.store_scatter(csrc2, [row, col], sl, mask=msk)
      plsc.store_scatter(cslot2, [row, col], slot, mask=msk)
      plsc.addupdate_scatter(deg_l, [slot // DW, slot % DW], ones_f,
                             mask=msk)
      return m + lax.index_in_dim(inc, L - 1, axis=0, keepdims=False)

    def compact2(i, m, par=par):
      g0 = 2 * i
      g1 = g0 + 1
      _, _, slot0, sl0, msk0, inc0 = half(g0, m, par)
      _, _, slot1, sl1, msk1, inc1 = half(g1, m, par)
      m = emit(m, slot0, sl0, msk0, inc0)
      m = emit(m, slot1, sl1, msk1, inc1)
      return m
    m = lax.fori_loop(0, psz // (2 * L), compact2, m)

    mch = m // KC
    flush(f, mch)
    f = mch

  for j in range(KC // L):
    pos = m + j * L + iota
    plsc.store_scatter(csrc2, [(pos // KC) % TROWS, pos % KC],
                       jnp.zeros((L,), jnp.int32))
    plsc.store_scatter(cslot2, [(pos // KC) % TROWS, pos % KC],
                       jnp.full((L,), DUMMY, jnp.int32))
  flush(f, (m + (KC - 1)) // KC)

  pltpu.sync_copy(deg_l, deg_sh.at[iota_v], add=True)
  sc_work.__exit__(None, None, None)

  sc_bar2 = jax.named_scope("sc_bar2")
  sc_bar2.__enter__()
  plsc.subcore_barrier()
  sc_bar2.__exit__(None, None, None)

  sc_expand = jax.named_scope("sc_expand")
  sc_expand.__enter__()
  tbase = sid * BT
  pltpu.sync_copy(deg_sh, deg_l)
  def sexp(g, _):
    idx = node_f[pl.ds(tbase + g * L, L)]
    slot = plsc.load_gather(smap_v, [idx])
    slotidx_v[pl.ds(g * L, L)] = slot
    degout_v[pl.ds(g * L, L)] = plsc.load_gather(
        deg_l, [slot // DW, slot % DW])
    return 0
  lax.fori_loop(0, BT // L, sexp, 0)

  for q in range(BT // KC):
    pltpu.async_copy(acc_sh.at[slotidx_v.at[pl.ds(q * KC, KC)]],
                     rows_a, gsem_a).wait()
    pltpu.sync_copy(rows_a,
                    acc_out.at[pl.ds(cid * B + tbase + q * KC, KC)])
  pltpu.sync_copy(degout_v, deg_out.at[pl.ds(cid * B + tbase, BT)])
  sc_expand.__exit__(None, None, None)


def _make_sc():
  mesh = plsc.VectorSubcoreMesh(core_axis_name="c", subcore_axis_name="s")
  return pl.kernel(
      _sc_kernel,
      out_type=[jax.ShapeDtypeStruct((NC * B, D), jnp.float32),
                jax.ShapeDtypeStruct((NC * B,), jnp.float32)],
      mesh=mesh,
      compiler_params=pltpu.CompilerParams(needs_layout_passes=False),
      scratch_types=[
          pltpu.VMEM((EPC,), jnp.int32),
          pltpu.VMEM((EPC,), jnp.int32),
          pltpu.VMEM((EPC,), jnp.int32),
          pltpu.VMEM((EPC,), jnp.int32),
          pltpu.VMEM((B,), jnp.int32),
          pltpu.VMEM((N,), jnp.int32),
          pltpu.VMEM((DROWS, DW), jnp.float32),
          pltpu.VMEM((TROWS, KC), jnp.int32),
          pltpu.VMEM((TROWS, KC), jnp.int32),
          pltpu.VMEM((KC, D), jnp.float32),
          pltpu.VMEM((KC, D), jnp.float32),
          pltpu.VMEM((DROWS,), jnp.int32),
          pltpu.VMEM((BT,), jnp.int32),
          pltpu.VMEM((BT,), jnp.float32),
          pltpu.VMEM_SHARED((HALF, D), jnp.float32),
          pltpu.VMEM_SHARED((SLOTS, D), jnp.float32),
          pltpu.VMEM_SHARED((DROWS, DW), jnp.float32),
          pltpu.VMEM_SHARED((N,), jnp.int32),
          pltpu.SemaphoreType.DMA,
          pltpu.SemaphoreType.DMA,
          pltpu.SemaphoreType.DMA,
          pltpu.SemaphoreType.DMA,
      ],
  )


def _tc_head(acc_ref, deg_ref, wg_ref, bg_ref, wm_ref, bm_ref, out_ref):
  a = acc_ref[pl.ds(0, B), :] + acc_ref[pl.ds(B, B), :]
  d = jnp.sum(deg_ref[...], axis=1, keepdims=True)
  h = a / jnp.maximum(d, 1.0)
  r = jnp.maximum(jnp.dot(h, wg_ref[...],
                          preferred_element_type=jnp.float32) + bg_ref[...], 0.0)
  out_ref[...] = jnp.dot(r, wm_ref[...],
                         preferred_element_type=jnp.float32) + bm_ref[...]


def kernel(x, edge_index, node, input, W_gnn, b_gnn, W_mlp, b_mlp):
  del input
  acc, deg = _make_sc()(x, edge_index.reshape(-1), node)
  degt = deg.reshape(NC, B).T
  out = pl.pallas_call(
      _tc_head,
      out_shape=jax.ShapeDtypeStruct((B, C), jnp.float32),
  )(acc, degt, W_gnn, b_gnn.reshape(1, D), W_mlp, b_mlp.reshape(1, C))
  return out


# speedup vs baseline: 1.0244x; 1.0244x over previous
import jax
import jax.numpy as jnp
from jax import lax
from jax.experimental import pallas as pl
from jax.experimental.pallas import tpu as pltpu
from jax.experimental.pallas import tpu_sc as plsc

N = 10000
E = 320000
D = 128
C = 40
B = 2048

NC = 2
NS = 16
L = 16

HALF = N // NC
EPT = E // NS
EPC = 2048
PASS_SIZES = [EPC] * (EPT // EPC) + [EPT - (EPT // EPC) * EPC]
KC = 32
TROWS = 128
SLOTS = 2176
DUMMY = B
DW = 128
DROWS = 32
BT = B // NS
XPT = 320
MPT = 624


def _sc_kernel(x_hbm, ei_hbm, node_hbm,
               acc_out, deg_out,
               edb_s0, edb_s1, edb_d0, edb_d1, node_f, smap_v, deg_l,
               csrc2, cslot2, rows_a, rows_b,
               iota_v, slotidx_v, degout_v,
               x_sh, acc_sh, deg_sh, smap_sh,
               gsem_a, gsem_b, esem, xsem):
  cid = lax.axis_index("c")
  sid = lax.axis_index("s")
  ebase = sid * EPT
  xbase = cid * HALF

  sc_setup = jax.named_scope("sc_setup")
  sc_setup.__enter__()
  edbs = (edb_s0, edb_s1)
  edbd = (edb_d0, edb_d1)
  pltpu.async_copy(ei_hbm.at[pl.ds(ebase, EPC)], edb_s0, esem)
  pltpu.async_copy(ei_hbm.at[pl.ds(E + ebase, EPC)], edb_d0, esem)
  @pl.when(sid < NS - 1)
  def _xstage():
    pltpu.async_copy(x_hbm.at[pl.ds(xbase + sid * XPT, XPT)],
                     x_sh.at[pl.ds(sid * XPT, XPT)], xsem)
  @pl.when(sid == NS - 1)
  def _xstage_last():
    pltpu.async_copy(
        x_hbm.at[pl.ds(xbase + (NS - 1) * XPT, HALF - (NS - 1) * XPT)],
        x_sh.at[pl.ds((NS - 1) * XPT, HALF - (NS - 1) * XPT)], xsem)

  zeros_f = jnp.zeros((L,), jnp.float32)
  ones_f = jnp.ones((L,), jnp.float32)
  iota = lax.iota(jnp.int32, L)

  def zrows(i, _):
    for j in range(D // L):
      rows_a[i, pl.ds(j * L, L)] = zeros_f
    return 0
  lax.fori_loop(0, KC, zrows, 0)

  def zdl(i, _):
    for j in range(DW // L):
      deg_l[i, pl.ds(j * L, L)] = zeros_f
    return 0
  lax.fori_loop(0, DROWS, zdl, 0)

  for g in range(DROWS // L):
    iota_v[pl.ds(g * L, L)] = g * L + iota

  zbase = sid * (SLOTS // NS)
  for q in range(4):
    pltpu.sync_copy(rows_a, acc_sh.at[pl.ds(zbase + q * KC, KC)])
  pltpu.sync_copy(rows_a.at[pl.ds(0, SLOTS // NS - 4 * KC)],
                  acc_sh.at[pl.ds(zbase + 4 * KC, SLOTS // NS - 4 * KC)])

  @pl.when(sid == 0)
  def _zdeg():
    pltpu.sync_copy(rows_a, deg_sh)
  pltpu.sync_copy(node_hbm, node_f)
  mlo = sid * MPT
  msz = jnp.where(sid == NS - 1, N - (NS - 1) * MPT, MPT)
  neg = jnp.full((L,), -1, jnp.int32)
  def zmap(i, _):
    smap_v[pl.ds(mlo + i * L, L)] = neg
    return 0
  lax.fori_loop(0, msz // L, zmap, 0)
  def setmap(i, _):
    idx = node_f[pl.ds(i * L, L)]
    inrange = (idx >= mlo) & (idx < mlo + msz)
    plsc.store_scatter(smap_v, [idx], i * L + iota, mask=inrange)
    return 0
  lax.fori_loop(0, B // L, setmap, 0)
  pltpu.sync_copy(smap_v.at[pl.ds(mlo, 624)], smap_sh.at[pl.ds(mlo, 624)])
  @pl.when(sid == NS - 1)
  def _pub_tail():
    pltpu.sync_copy(smap_v.at[pl.ds(N - L, L)], smap_sh.at[pl.ds(N - L, L)])

  @pl.when(sid < NS - 1)
  def _xwait():
    pltpu.make_async_copy(x_hbm.at[pl.ds(xbase + sid * XPT, XPT)],
                          x_sh.at[pl.ds(sid * XPT, XPT)], xsem).wait()
  @pl.when(sid == NS - 1)
  def _xwait_last():
    pltpu.make_async_copy(
        x_hbm.at[pl.ds(xbase + (NS - 1) * XPT, HALF - (NS - 1) * XPT)],
        x_sh.at[pl.ds((NS - 1) * XPT, HALF - (NS - 1) * XPT)], xsem).wait()
  sc_setup.__exit__(None, None, None)
  sc_bar1 = jax.named_scope("sc_bar1")
  sc_bar1.__enter__()
  plsc.subcore_barrier()
  pltpu.sync_copy(smap_sh, smap_v)
  sc_bar1.__exit__(None, None, None)

  def flush(f, mch):
    fl = jax.named_scope("sc_flush")
    fl.__enter__()
    @pl.when(mch > f)
    def _prime():
      pltpu.async_copy(x_sh.at[csrc2.at[f % TROWS]], rows_a, gsem_a)
    def rowpair(i, _):
      ch0 = f + 2 * i
      ch1 = ch0 + 1
      r0 = ch0 % TROWS
      r1 = ch1 % TROWS
      pltpu.make_async_copy(x_sh.at[csrc2.at[r0]], rows_a, gsem_a).wait()
      @pl.when(ch1 < mch)
      def _g1():
        pltpu.async_copy(x_sh.at[csrc2.at[r1]], rows_b, gsem_b)
      pltpu.sync_copy(rows_a, acc_sh.at[cslot2.at[r0]], add=True)
      @pl.when(ch1 < mch)
      def _s1():
        pltpu.make_async_copy(x_sh.at[csrc2.at[r1]], rows_b, gsem_b).wait()
        @pl.when(ch1 + 1 < mch)
        def _g2():
          pltpu.async_copy(x_sh.at[csrc2.at[(ch1 + 1) % TROWS]],
                           rows_a, gsem_a)
        pltpu.sync_copy(rows_b, acc_sh.at[cslot2.at[r1]], add=True)
      return 0
    lax.fori_loop(0, (mch - f + 1) // 2, rowpair, 0)
    fl.__exit__(None, None, None)

  sc_work = jax.named_scope("sc_work")
  sc_work.__enter__()
  m = jnp.int32(0)
  f = jnp.int32(0)
  for p, psz in enumerate(PASS_SIZES):
    par = p % 2
    eoff = ebase + p * EPC
    pltpu.make_async_copy(ei_hbm.at[pl.ds(eoff, psz)],
                          edbs[par].at[pl.ds(0, psz)], esem).wait()
    pltpu.make_async_copy(ei_hbm.at[pl.ds(E + eoff, psz)],
                          edbd[par].at[pl.ds(0, psz)], esem).wait()
    if p + 1 < len(PASS_SIZES):
      nsz = PASS_SIZES[p + 1]
      pltpu.async_copy(ei_hbm.at[pl.ds(eoff + EPC, nsz)],
                       edbs[1 - par].at[pl.ds(0, nsz)], esem)
      pltpu.async_copy(ei_hbm.at[pl.ds(E + eoff + EPC, nsz)],
                       edbd[1 - par].at[pl.ds(0, nsz)], esem)

    def half(g, m, par):
      d = edbd[par][pl.ds(g * L, L)]
      s = edbs[par][pl.ds(g * L, L)]
      slot = plsc.load_gather(smap_v, [d])
      sl = s - xbase
      msk = (slot >= 0) & (sl >= 0) & (sl < HALF)
      mv = jnp.where(msk, 1, 0).astype(jnp.int32)
      inc = plsc.cumsum(mv)
      return d, s, slot, sl, msk, inc

    def emit(m, slot, sl, msk, inc):
      pos = m + inc - 1
      row = (pos // KC) % TROWS
      col = pos % KC
      plsc.store_scatter(csrc2, [row, col], sl, mask=msk)
      plsc.store_scatter(cslot2, [row, col], slot, mask=msk)
      plsc.addupdate_scatter(deg_l, [slot // DW, slot % DW], ones_f,
                             mask=msk)
      return m + lax.index_in_dim(inc, L - 1, axis=0, keepdims=False)

    def compact2(i, m, par=par):
      g0 = 2 * i
      g1 = g0 + 1
      _, _, slot0, sl0, msk0, inc0 = half(g0, m, par)
      _, _, slot1, sl1, msk1, inc1 = half(g1, m, par)
      m = emit(m, slot0, sl0, msk0, inc0)
      m = emit(m, slot1, sl1, msk1, inc1)
      return m
    m = lax.fori_loop(0, psz // (2 * L), compact2, m)

    mch = m // KC
    flush(f, mch)
    f = mch

  for j in range(KC // L):
    pos = m + j * L + iota
    plsc.store_scatter(csrc2, [(pos // KC) % TROWS, pos % KC],
                       jnp.zeros((L,), jnp.int32))
    plsc.store_scatter(cslot2, [(pos // KC) % TROWS, pos % KC],
                       jnp.full((L,), DUMMY, jnp.int32))
  flush(f, (m + (KC - 1)) // KC)

  pltpu.sync_copy(deg_l, deg_sh.at[iota_v], add=True)
  sc_work.__exit__(None, None, None)

  sc_bar2 = jax.named_scope("sc_bar2")
  sc_bar2.__enter__()
  plsc.subcore_barrier()
  sc_bar2.__exit__(None, None, None)

  sc_expand = jax.named_scope("sc_expand")
  sc_expand.__enter__()
  tbase = sid * BT
  pltpu.sync_copy(deg_sh, deg_l)
  def sexp(g, _):
    idx = node_f[pl.ds(tbase + g * L, L)]
    slot = plsc.load_gather(smap_v, [idx])
    slotidx_v[pl.ds(g * L, L)] = slot
    degout_v[pl.ds(g * L, L)] = plsc.load_gather(
        deg_l, [slot // DW, slot % DW])
    return 0
  lax.fori_loop(0, BT // L, sexp, 0)

  for q in range(BT // KC):
    pltpu.async_copy(acc_sh.at[slotidx_v.at[pl.ds(q * KC, KC)]],
                     rows_a, gsem_a).wait()
    pltpu.sync_copy(rows_a,
                    acc_out.at[pl.ds(cid * B + tbase + q * KC, KC)])
  pltpu.sync_copy(degout_v, deg_out.at[pl.ds(cid * B + tbase, BT)])
  sc_expand.__exit__(None, None, None)


def _make_sc():
  mesh = plsc.VectorSubcoreMesh(core_axis_name="c", subcore_axis_name="s")
  return pl.kernel(
      _sc_kernel,
      out_type=[jax.ShapeDtypeStruct((NC * B, D), jnp.float32),
                jax.ShapeDtypeStruct((NC * B,), jnp.float32)],
      mesh=mesh,
      compiler_params=pltpu.CompilerParams(needs_layout_passes=False),
      scratch_types=[
          pltpu.VMEM((EPC,), jnp.int32),
          pltpu.VMEM((EPC,), jnp.int32),
          pltpu.VMEM((EPC,), jnp.int32),
          pltpu.VMEM((EPC,), jnp.int32),
          pltpu.VMEM((B,), jnp.int32),
          pltpu.VMEM((N,), jnp.int32),
          pltpu.VMEM((DROWS, DW), jnp.float32),
          pltpu.VMEM((TROWS, KC), jnp.int32),
          pltpu.VMEM((TROWS, KC), jnp.int32),
          pltpu.VMEM((KC, D), jnp.float32),
          pltpu.VMEM((KC, D), jnp.float32),
          pltpu.VMEM((DROWS,), jnp.int32),
          pltpu.VMEM((BT,), jnp.int32),
          pltpu.VMEM((BT,), jnp.float32),
          pltpu.VMEM_SHARED((HALF, D), jnp.float32),
          pltpu.VMEM_SHARED((SLOTS, D), jnp.float32),
          pltpu.VMEM_SHARED((DROWS, DW), jnp.float32),
          pltpu.VMEM_SHARED((N,), jnp.int32),
          pltpu.SemaphoreType.DMA,
          pltpu.SemaphoreType.DMA,
          pltpu.SemaphoreType.DMA,
          pltpu.SemaphoreType.DMA,
      ],
  )


def _tc_head(acc_ref, deg_ref, wg_ref, bg_ref, wm_ref, bm_ref, out_ref):
  a = acc_ref[pl.ds(0, B), :] + acc_ref[pl.ds(B, B), :]
  d = jnp.sum(deg_ref[...], axis=1, keepdims=True)
  h = a / jnp.maximum(d, 1.0)
  r = jnp.maximum(jnp.dot(h, wg_ref[...],
                          preferred_element_type=jnp.float32) + bg_ref[...], 0.0)
  out_ref[...] = jnp.dot(r, wm_ref[...],
                         preferred_element_type=jnp.float32) + bm_ref[...]


def kernel(x, edge_index, node, input, W_gnn, b_gnn, W_mlp, b_mlp):
  del input
  acc, deg = _make_sc()(x, edge_index.reshape(-1), node)
  degt = deg.reshape(NC, B).T
  out = pl.pallas_call(
      _tc_head,
      out_shape=jax.ShapeDtypeStruct((B, C), jnp.float32),
  )(acc, degt, W_gnn, b_gnn.reshape(1, D), W_mlp, b_mlp.reshape(1, C))
  return out


# speedup vs baseline: 1.1946x; 1.1661x over previous
import jax
import jax.numpy as jnp
from jax import lax
from jax.experimental import pallas as pl
from jax.experimental.pallas import tpu as pltpu
from jax.experimental.pallas import tpu_sc as plsc

N = 10000
E = 320000
D = 128
C = 40
B = 2048

NC = 2
NS = 16
L = 16

HALF = N // NC
EPT = E // NS
EPC = 2048
PASS_SIZES = [EPC] * (EPT // EPC) + [EPT - (EPT // EPC) * EPC]
KC = 32
TROWS = 128
SLOTS = 2176
DUMMY = B
DW = 128
DROWS = 32
BT = B // NS
XPT = 320
MPT = 624


def _sc_kernel(x_hbm, ei_hbm, node_hbm,
               acc_out, deg_out,
               edb_s0, edb_s1, edb_d0, edb_d1, node_f, smap_v, deg_l,
               csrc2, cslot2, rows_a, rows_b,
               iota_v, slotidx_v, degout_v,
               x_sh, acc_sh, deg_sh, smap_sh,
               gsem_a, gsem_b, esem, xsem):
  cid = lax.axis_index("c")
  sid = lax.axis_index("s")
  ebase = sid * EPT
  xbase = cid * HALF

  sc_setup = jax.named_scope("sc_setup")
  sc_setup.__enter__()
  edbs = (edb_s0, edb_s1)
  edbd = (edb_d0, edb_d1)
  pltpu.async_copy(ei_hbm.at[pl.ds(ebase, EPC)], edb_s0, esem)
  pltpu.async_copy(ei_hbm.at[pl.ds(E + ebase, EPC)], edb_d0, esem)
  @pl.when(sid < NS - 1)
  def _xstage():
    pltpu.async_copy(x_hbm.at[pl.ds(xbase + sid * XPT, XPT)],
                     x_sh.at[pl.ds(sid * XPT, XPT)], xsem)
  @pl.when(sid == NS - 1)
  def _xstage_last():
    pltpu.async_copy(
        x_hbm.at[pl.ds(xbase + (NS - 1) * XPT, HALF - (NS - 1) * XPT)],
        x_sh.at[pl.ds((NS - 1) * XPT, HALF - (NS - 1) * XPT)], xsem)

  zeros_f = jnp.zeros((L,), jnp.float32)
  ones_f = jnp.ones((L,), jnp.float32)
  iota = lax.iota(jnp.int32, L)

  def zrows(i, _):
    for j in range(D // L):
      rows_a[i, pl.ds(j * L, L)] = zeros_f
    return 0
  lax.fori_loop(0, KC, zrows, 0)

  def zdl(i, _):
    for j in range(DW // L):
      deg_l[i, pl.ds(j * L, L)] = zeros_f
    return 0
  lax.fori_loop(0, DROWS, zdl, 0)

  for g in range(DROWS // L):
    iota_v[pl.ds(g * L, L)] = g * L + iota

  zbase = sid * (SLOTS // NS)
  for q in range(4):
    pltpu.sync_copy(rows_a, acc_sh.at[pl.ds(zbase + q * KC, KC)])
  pltpu.sync_copy(rows_a.at[pl.ds(0, SLOTS // NS - 4 * KC)],
                  acc_sh.at[pl.ds(zbase + 4 * KC, SLOTS // NS - 4 * KC)])

  @pl.when(sid == 0)
  def _zdeg():
    pltpu.sync_copy(rows_a, deg_sh)
  pltpu.sync_copy(node_hbm, node_f)
  mlo = sid * MPT
  msz = jnp.where(sid == NS - 1, N - (NS - 1) * MPT, MPT)
  neg = jnp.full((L,), -1, jnp.int32)
  def zmap(i, _):
    smap_v[pl.ds(mlo + i * L, L)] = neg
    return 0
  lax.fori_loop(0, msz // L, zmap, 0)
  def setmap(i, _):
    idx = node_f[pl.ds(i * L, L)]
    inrange = (idx >= mlo) & (idx < mlo + msz)
    plsc.store_scatter(smap_v, [idx], i * L + iota, mask=inrange)
    return 0
  lax.fori_loop(0, B // L, setmap, 0)
  pltpu.sync_copy(smap_v.at[pl.ds(mlo, 624)], smap_sh.at[pl.ds(mlo, 624)])
  @pl.when(sid == NS - 1)
  def _pub_tail():
    pltpu.sync_copy(smap_v.at[pl.ds(N - L, L)], smap_sh.at[pl.ds(N - L, L)])

  @pl.when(sid < NS - 1)
  def _xwait():
    pltpu.make_async_copy(x_hbm.at[pl.ds(xbase + sid * XPT, XPT)],
                          x_sh.at[pl.ds(sid * XPT, XPT)], xsem).wait()
  @pl.when(sid == NS - 1)
  def _xwait_last():
    pltpu.make_async_copy(
        x_hbm.at[pl.ds(xbase + (NS - 1) * XPT, HALF - (NS - 1) * XPT)],
        x_sh.at[pl.ds((NS - 1) * XPT, HALF - (NS - 1) * XPT)], xsem).wait()
  sc_setup.__exit__(None, None, None)
  sc_bar1 = jax.named_scope("sc_bar1")
  sc_bar1.__enter__()
  plsc.subcore_barrier()
  pltpu.sync_copy(smap_sh, smap_v)
  sc_bar1.__exit__(None, None, None)

  def flush(f, mch):
    fl = jax.named_scope("sc_flush")
    fl.__enter__()
    @pl.when(mch > f)
    def _prime():
      pltpu.async_copy(x_sh.at[csrc2.at[f % TROWS]], rows_a, gsem_a)
    def rowpair(i, _):
      ch0 = f + 2 * i
      ch1 = ch0 + 1
      r0 = ch0 % TROWS
      r1 = ch1 % TROWS
      pltpu.make_async_copy(x_sh.at[csrc2.at[r0]], rows_a, gsem_a).wait()
      @pl.when(ch1 < mch)
      def _g1():
        pltpu.async_copy(x_sh.at[csrc2.at[r1]], rows_b, gsem_b)
      pltpu.sync_copy(rows_a, acc_sh.at[cslot2.at[r0]], add=True)
      @pl.when(ch1 < mch)
      def _s1():
        pltpu.make_async_copy(x_sh.at[csrc2.at[r1]], rows_b, gsem_b).wait()
        @pl.when(ch1 + 1 < mch)
        def _g2():
          pltpu.async_copy(x_sh.at[csrc2.at[(ch1 + 1) % TROWS]],
                           rows_a, gsem_a)
        pltpu.sync_copy(rows_b, acc_sh.at[cslot2.at[r1]], add=True)
      return 0
    lax.fori_loop(0, (mch - f + 1) // 2, rowpair, 0)
    fl.__exit__(None, None, None)

  sc_work = jax.named_scope("sc_work")
  sc_work.__enter__()
  m = jnp.int32(0)
  f = jnp.int32(0)
  for p, psz in enumerate(PASS_SIZES):
    par = p % 2
    eoff = ebase + p * EPC
    pltpu.make_async_copy(ei_hbm.at[pl.ds(eoff, psz)],
                          edbs[par].at[pl.ds(0, psz)], esem).wait()
    pltpu.make_async_copy(ei_hbm.at[pl.ds(E + eoff, psz)],
                          edbd[par].at[pl.ds(0, psz)], esem).wait()
    if p + 1 < len(PASS_SIZES):
      nsz = PASS_SIZES[p + 1]
      pltpu.async_copy(ei_hbm.at[pl.ds(eoff + EPC, nsz)],
                       edbs[1 - par].at[pl.ds(0, nsz)], esem)
      pltpu.async_copy(ei_hbm.at[pl.ds(E + eoff + EPC, nsz)],
                       edbd[1 - par].at[pl.ds(0, nsz)], esem)

    def half(g, m, par):
      d = edbd[par][pl.ds(g * L, L)]
      s = edbs[par][pl.ds(g * L, L)]
      slot = plsc.load_gather(smap_v, [d])
      sl = s - xbase
      msk = (slot >= 0) & (sl >= 0) & (sl < HALF)
      mv = jnp.where(msk, 1, 0).astype(jnp.int32)
      inc = plsc.cumsum(mv)
      return d, s, slot, sl, msk, inc

    def emit(m, slot, sl, msk, inc):
      pos = m + inc - 1
      row = (pos // KC) % TROWS
      col = pos % KC
      plsc.store_scatter(csrc2, [row, col], sl, mask=msk)
      plsc.store_scatter(cslot2, [row, col], slot, mask=msk)
      plsc.addupdate_scatter(deg_l, [slot // DW, slot % DW], ones_f,
                             mask=msk)
      return m + lax.index_in_dim(inc, L - 1, axis=0, keepdims=False)

    @plsc.parallel_loop(0, psz // (2 * L), unroll=2, carry=m)
    def _cloop(i, mm, par=par):
      g0 = 2 * i
      g1 = g0 + 1
      _, _, slot0, sl0, msk0, inc0 = half(g0, mm, par)
      _, _, slot1, sl1, msk1, inc1 = half(g1, mm, par)
      mm = emit(mm, slot0, sl0, msk0, inc0)
      mm = emit(mm, slot1, sl1, msk1, inc1)
      return mm
    m = _cloop

    mch = m // KC
    flush(f, mch)
    f = mch

  for j in range(KC // L):
    pos = m + j * L + iota
    plsc.store_scatter(csrc2, [(pos // KC) % TROWS, pos % KC],
                       jnp.zeros((L,), jnp.int32))
    plsc.store_scatter(cslot2, [(pos // KC) % TROWS, pos % KC],
                       jnp.full((L,), DUMMY, jnp.int32))
  flush(f, (m + (KC - 1)) // KC)

  pltpu.sync_copy(deg_l, deg_sh.at[iota_v], add=True)
  sc_work.__exit__(None, None, None)

  sc_bar2 = jax.named_scope("sc_bar2")
  sc_bar2.__enter__()
  plsc.subcore_barrier()
  sc_bar2.__exit__(None, None, None)

  sc_expand = jax.named_scope("sc_expand")
  sc_expand.__enter__()
  tbase = sid * BT
  pltpu.sync_copy(deg_sh, deg_l)
  def sexp(g, _):
    idx = node_f[pl.ds(tbase + g * L, L)]
    slot = plsc.load_gather(smap_v, [idx])
    slotidx_v[pl.ds(g * L, L)] = slot
    degout_v[pl.ds(g * L, L)] = plsc.load_gather(
        deg_l, [slot // DW, slot % DW])
    return 0
  lax.fori_loop(0, BT // L, sexp, 0)

  for q in range(BT // KC):
    pltpu.async_copy(acc_sh.at[slotidx_v.at[pl.ds(q * KC, KC)]],
                     rows_a, gsem_a).wait()
    pltpu.sync_copy(rows_a,
                    acc_out.at[pl.ds(cid * B + tbase + q * KC, KC)])
  pltpu.sync_copy(degout_v, deg_out.at[pl.ds(cid * B + tbase, BT)])
  sc_expand.__exit__(None, None, None)


def _make_sc():
  mesh = plsc.VectorSubcoreMesh(core_axis_name="c", subcore_axis_name="s")
  return pl.kernel(
      _sc_kernel,
      out_type=[jax.ShapeDtypeStruct((NC * B, D), jnp.float32),
                jax.ShapeDtypeStruct((NC * B,), jnp.float32)],
      mesh=mesh,
      compiler_params=pltpu.CompilerParams(needs_layout_passes=False),
      scratch_types=[
          pltpu.VMEM((EPC,), jnp.int32),
          pltpu.VMEM((EPC,), jnp.int32),
          pltpu.VMEM((EPC,), jnp.int32),
          pltpu.VMEM((EPC,), jnp.int32),
          pltpu.VMEM((B,), jnp.int32),
          pltpu.VMEM((N,), jnp.int32),
          pltpu.VMEM((DROWS, DW), jnp.float32),
          pltpu.VMEM((TROWS, KC), jnp.int32),
          pltpu.VMEM((TROWS, KC), jnp.int32),
          pltpu.VMEM((KC, D), jnp.float32),
          pltpu.VMEM((KC, D), jnp.float32),
          pltpu.VMEM((DROWS,), jnp.int32),
          pltpu.VMEM((BT,), jnp.int32),
          pltpu.VMEM((BT,), jnp.float32),
          pltpu.VMEM_SHARED((HALF, D), jnp.float32),
          pltpu.VMEM_SHARED((SLOTS, D), jnp.float32),
          pltpu.VMEM_SHARED((DROWS, DW), jnp.float32),
          pltpu.VMEM_SHARED((N,), jnp.int32),
          pltpu.SemaphoreType.DMA,
          pltpu.SemaphoreType.DMA,
          pltpu.SemaphoreType.DMA,
          pltpu.SemaphoreType.DMA,
      ],
  )


def _tc_head(acc_ref, deg_ref, wg_ref, bg_ref, wm_ref, bm_ref, out_ref):
  a = acc_ref[pl.ds(0, B), :] + acc_ref[pl.ds(B, B), :]
  d = jnp.sum(deg_ref[...], axis=1, keepdims=True)
  h = a / jnp.maximum(d, 1.0)
  r = jnp.maximum(jnp.dot(h, wg_ref[...],
                          preferred_element_type=jnp.float32) + bg_ref[...], 0.0)
  out_ref[...] = jnp.dot(r, wm_ref[...],
                         preferred_element_type=jnp.float32) + bm_ref[...]


def kernel(x, edge_index, node, input, W_gnn, b_gnn, W_mlp, b_mlp):
  del input
  acc, deg = _make_sc()(x, edge_index.reshape(-1), node)
  degt = deg.reshape(NC, B).T
  out = pl.pallas_call(
      _tc_head,
      out_shape=jax.ShapeDtypeStruct((B, C), jnp.float32),
  )(acc, degt, W_gnn, b_gnn.reshape(1, D), W_mlp, b_mlp.reshape(1, C))
  return out
